# Initial kernel scaffold; baseline (speedup 1.0000x reference)
import jax
import jax.numpy as jnp
from jax.experimental import pallas as pl


def kernel(x, enc_w0, enc_b0, enc_w1, enc_b1, enc_w2, enc_b2, enc_w3, enc_b3, enc_w4, enc_b4, dec_w0, dec_b0, dec_w1, dec_b1, dec_w2, dec_b2, dec_w3, dec_b3, out_w, out_b):
    raise NotImplementedError("write your pallas kernel here")


# speedup vs baseline: 414.1464x; 414.1464x over previous
import functools

import jax
import jax.numpy as jnp
from jax.experimental import pallas as pl


def _lap_valid(a):
    R = a.shape[0]
    acc = a[0:R - 8]
    for o in range(1, 9):
        acc = acc + a[o:R - 8 + o]
    return (a[4:R - 4] - acc) * 0.125


def _cheb_body(x_ref, w_ref, b_ref, *out_refs, cn, n, relu, pool):
    c = pl.program_id(1)
    nb = n // cn
    s = c * cn
    lo = jnp.where(c == 0, n - 8, s - 8)
    hi = jnp.where(c == nb - 1, 0, s + cn)
    win = jnp.concatenate(
        [x_ref[pl.ds(lo, 8), :], x_ref[pl.ds(s, cn), :], x_ref[pl.ds(hi, 8), :]],
        axis=0)
    x1w = _lap_valid(win)
    x0c = win[8:cn + 8]
    x1c = x1w[4:cn + 4]
    x2c = 2.0 * _lap_valid(x1w) - x0c
    acc = jnp.dot(x0c, w_ref[0], preferred_element_type=jnp.float32)
    acc = acc + jnp.dot(x1c, w_ref[1], preferred_element_type=jnp.float32)
    acc = acc + jnp.dot(x2c, w_ref[2], preferred_element_type=jnp.float32)
    acc = acc + b_ref[0, :][None, :]
    if relu:
        acc = jnp.maximum(acc, 0.0)
    out_refs[0][...] = acc
    if pool:
        fout = acc.shape[1]
        p = acc.reshape(cn // 4, 4, fout)
        out_refs[1][...] = (p[:, 0] + p[:, 1] + p[:, 2] + p[:, 3]) * 0.25


def _cheb_conv(x, w, b, *, relu, pool):
    bsz, n, fin = x.shape
    kk, _, fout = w.shape
    cn = min(n, 1024)
    nb = n // cn
    out_shape = [jax.ShapeDtypeStruct((bsz, n, fout), jnp.float32)]
    out_specs = [pl.BlockSpec((None, cn, fout), lambda bb, c: (bb, c, 0))]
    if pool:
        out_shape.append(jax.ShapeDtypeStruct((bsz, n // 4, fout), jnp.float32))
        out_specs.append(pl.BlockSpec((None, cn // 4, fout), lambda bb, c: (bb, c, 0)))
    body = functools.partial(_cheb_body, cn=cn, n=n, relu=relu, pool=pool)
    return pl.pallas_call(
        body,
        grid=(bsz, nb),
        in_specs=[
            pl.BlockSpec((None, n, fin), lambda bb, c: (bb, 0, 0)),
            pl.BlockSpec((kk, fin, fout), lambda bb, c: (0, 0, 0)),
            pl.BlockSpec((1, fout), lambda bb, c: (0, 0)),
        ],
        out_shape=out_shape,
        out_specs=out_specs,
    )(x, w, b.reshape(1, fout))


def kernel(x, enc_w0, enc_b0, enc_w1, enc_b1, enc_w2, enc_b2, enc_w3, enc_b3,
           enc_w4, enc_b4, dec_w0, dec_b0, dec_w1, dec_b1, dec_w2, dec_b2,
           dec_w3, dec_b3, out_w, out_b):
    enc = [(enc_w0, enc_b0), (enc_w1, enc_b1), (enc_w2, enc_b2),
           (enc_w3, enc_b3), (enc_w4, enc_b4)]
    dec = [(dec_w0, dec_b0), (dec_w1, dec_b1), (dec_w2, dec_b2),
           (dec_w3, dec_b3)]
    skips = []
    h = x
    for i in range(4):
        h_full, h = _cheb_conv(h, enc[i][0], enc[i][1], relu=True, pool=True)
        skips.append(h_full)
    h = _cheb_conv(h, enc[4][0], enc[4][1], relu=True, pool=False)[0]
    for i in range(4):
        u = jnp.repeat(h, 4, axis=1)
        hcat = jnp.concatenate([u, skips[3 - i]], axis=-1)
        h = _cheb_conv(hcat, dec[i][0], dec[i][1], relu=True, pool=False)[0]
    return _cheb_conv(h, out_w, out_b, relu=False, pool=False)[0]


# speedup vs baseline: 595.7636x; 1.4385x over previous
import functools

import jax
import jax.numpy as jnp
from jax.experimental import pallas as pl


def _lap_valid(a):
    R = a.shape[0]
    w2 = a[0:R - 1] + a[1:R]
    w4 = w2[0:R - 3] + w2[2:R - 1]
    w8 = w4[0:R - 7] + w4[4:R - 3]
    w9 = w8[0:R - 8] + a[8:R]
    return (a[4:R - 4] - w9) * 0.125


def _cheb_body(x_ref, w_ref, b_ref, *out_refs, cn, n, relu, pool, after,
               chunk_axis):
    c = pl.program_id(chunk_axis)
    nb = n // cn
    s = c * cn
    lo = jnp.where(c == 0, n - 8, s - 8)
    hi = jnp.where(c == nb - 1, 0, s + cn)
    win = jnp.concatenate(
        [x_ref[pl.ds(lo, 8), :], x_ref[pl.ds(s, cn), :], x_ref[pl.ds(hi, 8), :]],
        axis=0)
    if after:
        y0 = jnp.dot(win[8:cn + 8], w_ref[0], preferred_element_type=jnp.float32)
        y1 = jnp.dot(win[4:cn + 12], w_ref[1], preferred_element_type=jnp.float32)
        y2 = jnp.dot(win, w_ref[2], preferred_element_type=jnp.float32)
        t = _lap_valid(y2)
        acc = y0 - y2[8:cn + 8] + _lap_valid(y1 + 2.0 * t)
    else:
        x1w = _lap_valid(win)
        x0c = win[8:cn + 8]
        x2c = 2.0 * _lap_valid(x1w) - x0c
        acc = jnp.dot(x0c, w_ref[0], preferred_element_type=jnp.float32)
        acc = acc + jnp.dot(x1w[4:cn + 4], w_ref[1],
                            preferred_element_type=jnp.float32)
        acc = acc + jnp.dot(x2c, w_ref[2], preferred_element_type=jnp.float32)
    acc = acc + b_ref[0, :][None, :]
    if relu:
        acc = jnp.maximum(acc, 0.0)
    out_refs[0][...] = acc
    if pool:
        fout = acc.shape[1]
        p = acc.reshape(cn // 4, 4, fout)
        out_refs[1][...] = (p[:, 0] + p[:, 1] + p[:, 2] + p[:, 3]) * 0.25


def _pick_cn(n):
    if n <= 2048:
        return n
    return 2048 if n % 2048 == 0 else 1536


def _cheb_conv(x, w, b, *, relu, pool, after, packed):
    kk = w.shape[0]
    if packed:
        n, fin = x.shape
        fout = w.shape[2]
        cn = _pick_cn(n)
        nb = n // cn
        grid = (nb,)
        chunk_axis = 0
        in_specs = [
            pl.BlockSpec((n, fin), lambda c: (0, 0)),
            pl.BlockSpec((kk, fin, fout), lambda c: (0, 0, 0)),
            pl.BlockSpec((1, fout), lambda c: (0, 0)),
        ]
        out_shape = [jax.ShapeDtypeStruct((n, fout), jnp.float32)]
        out_specs = [pl.BlockSpec((cn, fout), lambda c: (c, 0))]
        if pool:
            out_shape.append(jax.ShapeDtypeStruct((n // 4, fout), jnp.float32))
            out_specs.append(pl.BlockSpec((cn // 4, fout), lambda c: (c, 0)))
    else:
        bsz, n, fin = x.shape
        fout = w.shape[2]
        cn = _pick_cn(n)
        nb = n // cn
        grid = (bsz, nb)
        chunk_axis = 1
        in_specs = [
            pl.BlockSpec((None, n, fin), lambda bb, c: (bb, 0, 0)),
            pl.BlockSpec((kk, fin, fout), lambda bb, c: (0, 0, 0)),
            pl.BlockSpec((1, fout), lambda bb, c: (0, 0)),
        ]
        out_shape = [jax.ShapeDtypeStruct((bsz, n, fout), jnp.float32)]
        out_specs = [pl.BlockSpec((None, cn, fout), lambda bb, c: (bb, c, 0))]
        if pool:
            out_shape.append(jax.ShapeDtypeStruct((bsz, n // 4, fout),
                                                  jnp.float32))
            out_specs.append(pl.BlockSpec((None, cn // 4, fout),
                                          lambda bb, c: (bb, c, 0)))
    body = functools.partial(_cheb_body, cn=cn, n=n, relu=relu, pool=pool,
                             after=after, chunk_axis=chunk_axis)
    return pl.pallas_call(
        body,
        grid=grid,
        in_specs=in_specs,
        out_shape=out_shape,
        out_specs=out_specs,
    )(x, w, b.reshape(1, fout))


def _pack(x):
    b, n, f = x.shape
    return jnp.transpose(x, (1, 0, 2)).reshape(n, b * f)


def _unpack(xp, f):
    n = xp.shape[0]
    return jnp.transpose(xp.reshape(n, 2, f), (1, 0, 2))


def _pack_w(w):
    eye = jnp.eye(2, dtype=w.dtype)
    return jnp.stack([jnp.kron(eye, w[k]) for k in range(w.shape[0])])


def _pack_b(b):
    return jnp.concatenate([b, b])


def kernel(x, enc_w0, enc_b0, enc_w1, enc_b1, enc_w2, enc_b2, enc_w3, enc_b3,
           enc_w4, enc_b4, dec_w0, dec_b0, dec_w1, dec_b1, dec_w2, dec_b2,
           dec_w3, dec_b3, out_w, out_b):
    h = _pack(x)
    skip0, h = _cheb_conv(h, _pack_w(enc_w0), _pack_b(enc_b0),
                          relu=True, pool=True, after=False, packed=True)
    skip1, h = _cheb_conv(h, _pack_w(enc_w1), _pack_b(enc_b1),
                          relu=True, pool=True, after=False, packed=True)
    skip2, h = _cheb_conv(h, _pack_w(enc_w2), _pack_b(enc_b2),
                          relu=True, pool=True, after=False, packed=True)
    h = _unpack(h, 128)
    skip3, h = _cheb_conv(h, enc_w3, enc_b3,
                          relu=True, pool=True, after=False, packed=False)
    h = _cheb_conv(h, enc_w4, enc_b4,
                   relu=True, pool=False, after=False, packed=False)[0]
    h = _cheb_conv(jnp.concatenate([jnp.repeat(h, 4, axis=1), skip3], axis=-1),
                   dec_w0, dec_b0, relu=True, pool=False, after=True,
                   packed=False)[0]
    h = _cheb_conv(jnp.concatenate([jnp.repeat(h, 4, axis=1),
                                    _unpack(skip2, 128)], axis=-1),
                   dec_w1, dec_b1, relu=True, pool=False, after=True,
                   packed=False)[0]
    up = jnp.repeat(_pack(h), 4, axis=0)
    xin = jnp.concatenate([up[:, :128], skip1[:, :64],
                           up[:, 128:], skip1[:, 64:]], axis=1)
    h = _cheb_conv(xin, _pack_w(dec_w2), _pack_b(dec_b2),
                   relu=True, pool=False, after=True, packed=True)[0]
    up = jnp.repeat(h, 4, axis=0)
    xin = jnp.concatenate([up[:, :64], skip0[:, :32],
                           up[:, 64:], skip0[:, 32:]], axis=1)
    h = _cheb_conv(xin, _pack_w(dec_w3), _pack_b(dec_b3),
                   relu=True, pool=False, after=True, packed=True)[0]
    outp = _cheb_conv(h, _pack_w(out_w), _pack_b(out_b),
                      relu=False, pool=False, after=True, packed=True)[0]
    return _unpack(outp, 2)
